# Initial kernel scaffold; baseline (speedup 1.0000x reference)
import jax
import jax.numpy as jnp
from jax.experimental import pallas as pl


def kernel(q, k, v):
    raise NotImplementedError("write your pallas kernel here")


# speedup vs baseline: 58.5064x; 58.5064x over previous
import functools

import jax
import jax.numpy as jnp
from jax.experimental import pallas as pl

_T = 2048
_D = 128
_ROW = 128
_NBLK = _T // _ROW


def _lv_body(q_ref, k_ref, v_ref, costs_ref, upd_ref, j_ref):
    f32 = jnp.float32
    lanes = jax.lax.broadcasted_iota(jnp.int32, (1, _ROW), 1)

    def _step(i, carry, blk):
        J, Sqq, SqvT, trSvv, cost_row, upd_row = carry
        t = blk * _ROW + i
        fl = (t + 1).astype(f32)
        dec = (fl - 1.0) / fl
        qr = q_ref[pl.ds(t, 1), :]
        kr = k_ref[pl.ds(t, 1), :]
        vr = v_ref[pl.ds(t, 1), :]
        qc = jnp.transpose(qr)
        vc = jnp.transpose(vr)

        Sqq = dec * Sqq + (qc * qr) / fl
        SqvT = dec * SqvT + (vc * qr) / fl
        vv = jnp.sum(vr * vr, keepdims=True)
        trSvv = dec * trSvv + vv / fl

        JS = jnp.dot(J, Sqq, preferred_element_type=f32)
        kS = jnp.dot(kr, Sqq, preferred_element_type=f32)
        ovk = vc * kr

        s_J = jnp.sum(J * SqvT, keepdims=True)[:1, :1]
        s_l = jnp.sum(SqvT * ovk, keepdims=True)[:1, :1]
        A_JJ = jnp.sum(JS * J, keepdims=True)[:1, :1]
        A_Jl = jnp.sum(JS * ovk, keepdims=True)[:1, :1]
        A_ll = vv * jnp.sum(kS * kr, keepdims=True)[:1, :1]

        first = t == 0
        A_JJ_s = jnp.where(first | (A_JJ == 0.0), 1.0, A_JJ)
        A_ll_s = jnp.where(first | (A_ll == 0.0), 1.0, A_ll)
        denom = A_JJ * A_ll - A_Jl * A_Jl
        denom_s = jnp.where(first | (denom == 0.0), 1.0, denom)
        margin = s_l - A_Jl * (s_J / A_JJ_s)
        wf = (A_ll * s_J - A_Jl * s_l) / denom_s
        wi = (A_JJ * s_l - A_Jl * s_J) / denom_s
        wf_c = jnp.where(wi <= 0.0, s_J / A_JJ_s, jnp.where(wf <= 0.0, 0.0, wf))
        wi_c = jnp.where(wi <= 0.0, 0.0, jnp.where(wf <= 0.0, s_l / A_ll_s, wi))
        do_upd = margin > 0.0
        wF = jnp.where(first, 0.0, jnp.where(do_upd, wf_c, 1.0))
        wI = jnp.where(first, 1.0, jnp.where(do_upd, wi_c, 0.0))
        J = wF * J + wI * ovk
        cost = (0.5 * trSvv - (wF * s_J + wI * s_l)
                + 0.5 * (wF * wF * A_JJ + 2.0 * wF * wI * A_Jl + wI * wI * A_ll))
        upd = jnp.where(first | do_upd, 1.0, 0.0)

        m = lanes == i
        cost_row = jnp.where(m, cost, cost_row)
        upd_row = jnp.where(m, upd, upd_row)
        return J, Sqq, SqvT, trSvv, cost_row, upd_row

    def _blk(blk, carry):
        J, Sqq, SqvT, trSvv = carry
        cost_row = jnp.zeros((1, _ROW), f32)
        upd_row = jnp.zeros((1, _ROW), f32)
        J, Sqq, SqvT, trSvv, cost_row, upd_row = jax.lax.fori_loop(
            0, _ROW, functools.partial(_step, blk=blk),
            (J, Sqq, SqvT, trSvv, cost_row, upd_row))
        costs_ref[pl.ds(blk, 1)] = cost_row.reshape(1, 1, _ROW)
        upd_ref[pl.ds(blk, 1)] = upd_row.reshape(1, 1, _ROW)
        return J, Sqq, SqvT, trSvv

    init = (jnp.zeros((_D, _D), f32), jnp.zeros((_D, _D), f32),
            jnp.zeros((_D, _D), f32), jnp.zeros((1, 1), f32))
    J, _, _, _ = jax.lax.fori_loop(0, _NBLK, _blk, init)
    j_ref[:, :] = J


def kernel(q, k, v):
    costs3, upd3, J = pl.pallas_call(
        _lv_body,
        out_shape=(
            jax.ShapeDtypeStruct((_NBLK, 1, _ROW), jnp.float32),
            jax.ShapeDtypeStruct((_NBLK, 1, _ROW), jnp.float32),
            jax.ShapeDtypeStruct((_D, _D), jnp.float32),
        ),
    )(q, k, v)
    return costs3.reshape(_T), upd3.reshape(_T) != 0.0, J


# speedup vs baseline: 75.6983x; 1.2938x over previous
import functools

import jax
import jax.numpy as jnp
from jax.experimental import pallas as pl

_T = 2048
_D = 128
_ROW = 128
_NBLK = _T // _ROW


def _lv_body(q_ref, k_ref, v_ref, costs_ref, upd_ref, j_ref):
    f32 = jnp.float32
    lanes = jax.lax.broadcasted_iota(jnp.int32, (1, _ROW), 1)

    def _step(i, carry, blk):
        J, Sqq, SqvT, trSvv, cost_row, upd_row = carry
        t = blk * _ROW + i
        fl = (t + 1).astype(f32)
        dec = (fl - 1.0) / fl
        qr = q_ref[pl.ds(t, 1), :]
        kr = k_ref[pl.ds(t, 1), :]
        vr = v_ref[pl.ds(t, 1), :]
        qc = jnp.transpose(qr)
        vc = jnp.transpose(vr)

        Sqq = dec * Sqq + (qc * qr) / fl
        SqvT = dec * SqvT + (vc * qr) / fl
        vv = jnp.sum(vr * vr, keepdims=True)
        trSvv = dec * trSvv + vv / fl

        JS = jnp.dot(J, Sqq, preferred_element_type=f32)
        kS = jnp.dot(kr, Sqq, preferred_element_type=f32)
        ovk = vc * kr

        s_J = jnp.sum(J * SqvT, keepdims=True)[:1, :1]
        s_l = jnp.sum(SqvT * ovk, keepdims=True)[:1, :1]
        A_JJ = jnp.sum(JS * J, keepdims=True)[:1, :1]
        A_Jl = jnp.sum(JS * ovk, keepdims=True)[:1, :1]
        A_ll = vv * jnp.sum(kS * kr, keepdims=True)[:1, :1]

        first = t == 0
        A_JJ_s = jnp.where(first | (A_JJ == 0.0), 1.0, A_JJ)
        A_ll_s = jnp.where(first | (A_ll == 0.0), 1.0, A_ll)
        denom = A_JJ * A_ll - A_Jl * A_Jl
        denom_s = jnp.where(first | (denom == 0.0), 1.0, denom)
        margin = s_l - A_Jl * (s_J / A_JJ_s)
        wf = (A_ll * s_J - A_Jl * s_l) / denom_s
        wi = (A_JJ * s_l - A_Jl * s_J) / denom_s
        wf_c = jnp.where(wi <= 0.0, s_J / A_JJ_s, jnp.where(wf <= 0.0, 0.0, wf))
        wi_c = jnp.where(wi <= 0.0, 0.0, jnp.where(wf <= 0.0, s_l / A_ll_s, wi))
        do_upd = margin > 0.0
        wF = jnp.where(first, 0.0, jnp.where(do_upd, wf_c, 1.0))
        wI = jnp.where(first, 1.0, jnp.where(do_upd, wi_c, 0.0))
        J = wF * J + wI * ovk
        cost = (0.5 * trSvv - (wF * s_J + wI * s_l)
                + 0.5 * (wF * wF * A_JJ + 2.0 * wF * wI * A_Jl + wI * wI * A_ll))
        upd = jnp.where(first | do_upd, 1.0, 0.0)

        m = lanes == i
        cost_row = jnp.where(m, cost, cost_row)
        upd_row = jnp.where(m, upd, upd_row)
        return J, Sqq, SqvT, trSvv, cost_row, upd_row

    def _blk(blk, carry):
        J, Sqq, SqvT, trSvv = carry
        cost_row = jnp.zeros((1, _ROW), f32)
        upd_row = jnp.zeros((1, _ROW), f32)
        J, Sqq, SqvT, trSvv, cost_row, upd_row = jax.lax.fori_loop(
            0, _ROW, functools.partial(_step, blk=blk),
            (J, Sqq, SqvT, trSvv, cost_row, upd_row), unroll=8)
        costs_ref[pl.ds(blk, 1)] = cost_row.reshape(1, 1, _ROW)
        upd_ref[pl.ds(blk, 1)] = upd_row.reshape(1, 1, _ROW)
        return J, Sqq, SqvT, trSvv

    init = (jnp.zeros((_D, _D), f32), jnp.zeros((_D, _D), f32),
            jnp.zeros((_D, _D), f32), jnp.zeros((1, 1), f32))
    J, _, _, _ = jax.lax.fori_loop(0, _NBLK, _blk, init)
    j_ref[:, :] = J


def kernel(q, k, v):
    costs3, upd3, J = pl.pallas_call(
        _lv_body,
        out_shape=(
            jax.ShapeDtypeStruct((_NBLK, 1, _ROW), jnp.float32),
            jax.ShapeDtypeStruct((_NBLK, 1, _ROW), jnp.float32),
            jax.ShapeDtypeStruct((_D, _D), jnp.float32),
        ),
    )(q, k, v)
    return costs3.reshape(_T), upd3.reshape(_T) != 0.0, J
